# Initial kernel scaffold; baseline (speedup 1.0000x reference)
import jax
import jax.numpy as jnp
from jax.experimental import pallas as pl


def kernel(x, edge_index, W1, b1, W2, b2):
    raise NotImplementedError("write your pallas kernel here")


# speedup vs baseline: 6.6057x; 6.6057x over previous
import functools

import jax
import jax.numpy as jnp
from jax import lax
from jax.experimental import pallas as pl
from jax.experimental.pallas import tpu as pltpu
from jax.experimental.pallas import tpu_sc as plsc

_N = 10000
_E = 320000
_D = 128

_NC = 2
_NS = 16
_NW = _NC * _NS
_CH = 128
_CPW = 80
_EP = _NW * _CPW * _CH
_RPT = 640
_NP = _NS * _RPT

_f32 = jnp.float32


def _sc_mesh():
    return plsc.VectorSubcoreMesh(core_axis_name="c", subcore_axis_name="s",
                                  num_cores=_NC, num_subcores=_NS)


def _fill_buf(buf, value):
    def _row(i, _):
        r = i // 8
        col = (i % 8) * 16
        buf[r, pl.ds(col, 16)] = jnp.full((16,), value, _f32)
        return _
    lax.fori_loop(0, _CH * 8, _row, None)


def _deg_body(dstp, out, buf, idx_d, acc):
    c = lax.axis_index("c")
    s = lax.axis_index("s")
    w = c * _NS + s

    _fill_buf(buf, 0.0)
    for j in range(_RPT // _CH):
        pltpu.sync_copy(buf, acc.at[pl.ds(s * _RPT + j * _CH, _CH)])
    plsc.subcore_barrier()

    _fill_buf(buf, 1.0)

    base = w * _CPW * _CH

    def _chunk(k, _):
        off = base + k * _CH
        pltpu.sync_copy(dstp.at[pl.ds(off, _CH)], idx_d)
        pltpu.sync_copy(buf, acc.at[idx_d], add=True)
        return _
    lax.fori_loop(0, _CPW, _chunk, None)

    plsc.subcore_barrier()
    pltpu.sync_copy(acc.at[pl.ds(s * _RPT, _RPT)],
                    out.at[pl.ds(c * _NP + s * _RPT, _RPT)])


@functools.partial(
    pl.kernel,
    out_type=jax.ShapeDtypeStruct((_NC * _NP, _D), _f32),
    mesh=_sc_mesh(),
    scratch_types=[
        pltpu.VMEM((_CH, _D), _f32),
        pltpu.VMEM((_CH,), jnp.int32),
        pltpu.VMEM_SHARED((_NP, _D), _f32),
    ],
)
def _deg_kernel(dstp, out, buf, idx_d, acc):
    _deg_body(dstp, out, buf, idx_d, acc)


def _agg_body(tbl, srcp, dstp, out, rows, idx_s, idx_d, acc, sem):
    c = lax.axis_index("c")
    s = lax.axis_index("s")
    w = c * _NS + s

    def _zero_row(i, _):
        r = i // 8
        col = (i % 8) * 16
        rows[r, pl.ds(col, 16)] = jnp.zeros((16,), _f32)
        return _
    lax.fori_loop(0, _CH * 8, _zero_row, None)
    for j in range(_RPT // _CH):
        pltpu.sync_copy(rows, acc.at[pl.ds(s * _RPT + j * _CH, _CH)])
    plsc.subcore_barrier()

    base = w * _CPW * _CH

    def _chunk(k, _):
        off = base + k * _CH
        pltpu.sync_copy(srcp.at[pl.ds(off, _CH)], idx_s)
        pltpu.sync_copy(dstp.at[pl.ds(off, _CH)], idx_d)
        pltpu.async_copy(tbl.at[idx_s], rows, sem).wait()
        pltpu.sync_copy(rows, acc.at[idx_d], add=True)
        return _
    lax.fori_loop(0, _CPW, _chunk, None)

    plsc.subcore_barrier()
    pltpu.sync_copy(acc.at[pl.ds(s * _RPT, _RPT)],
                    out.at[pl.ds(c * _NP + s * _RPT, _RPT)])


@functools.partial(
    pl.kernel,
    out_type=jax.ShapeDtypeStruct((_NC * _NP, _D), _f32),
    mesh=_sc_mesh(),
    scratch_types=[
        pltpu.VMEM((_CH, _D), _f32),
        pltpu.VMEM((_CH,), jnp.int32),
        pltpu.VMEM((_CH,), jnp.int32),
        pltpu.VMEM_SHARED((_NP, _D), _f32),
        pltpu.SemaphoreType.DMA,
    ],
)
def _agg_kernel(tbl, srcp, dstp, out, rows, idx_s, idx_d, acc, sem):
    _agg_body(tbl, srcp, dstp, out, rows, idx_s, idx_d, acc, sem)


_BLK = 1000
_GRID = _N // _BLK


def _k2_body(x_ref, w1_ref, deg_ref, h1s_ref, dinv_ref):
    deg = deg_ref[:, 0:1] + deg_ref[:, 1:2] + 1.0
    dinv = lax.rsqrt(deg)
    dinvb = jnp.broadcast_to(dinv, (_BLK, _D))
    h = jnp.dot(x_ref[:, :], w1_ref[:, :], preferred_element_type=_f32)
    h1s_ref[:, :] = h * dinvb
    dinv_ref[:, :] = dinvb


def _k2(x, W1, deg2):
    return pl.pallas_call(
        _k2_body,
        grid=(_GRID,),
        in_specs=[
            pl.BlockSpec((_BLK, _D), lambda i: (i, 0)),
            pl.BlockSpec((_D, _D), lambda i: (0, 0)),
            pl.BlockSpec((_BLK, 2), lambda i: (i, 0)),
        ],
        out_specs=[
            pl.BlockSpec((_BLK, _D), lambda i: (i, 0)),
            pl.BlockSpec((_BLK, _D), lambda i: (i, 0)),
        ],
        out_shape=[
            jax.ShapeDtypeStruct((_N, _D), _f32),
            jax.ShapeDtypeStruct((_N, _D), _f32),
        ],
    )(x, W1, deg2)


def _k4_body(s0_ref, s1_ref, h1s_ref, dinv_ref, b1_ref, w2_ref, h2s_ref):
    z = (s0_ref[:, :] + s1_ref[:, :] + h1s_ref[:, :]) * dinv_ref[:, :]
    z = z + b1_ref[:, :]
    h2 = jnp.maximum(z, 0.0)
    h2s_ref[:, :] = jnp.dot(h2, w2_ref[:, :],
                            preferred_element_type=_f32) * dinv_ref[:, :]


def _k4(s0, s1, h1s, dinvb, b1, W2):
    return pl.pallas_call(
        _k4_body,
        grid=(_GRID,),
        in_specs=[
            pl.BlockSpec((_BLK, _D), lambda i: (i, 0)),
            pl.BlockSpec((_BLK, _D), lambda i: (i, 0)),
            pl.BlockSpec((_BLK, _D), lambda i: (i, 0)),
            pl.BlockSpec((_BLK, _D), lambda i: (i, 0)),
            pl.BlockSpec((1, _D), lambda i: (0, 0)),
            pl.BlockSpec((_D, _D), lambda i: (0, 0)),
        ],
        out_specs=pl.BlockSpec((_BLK, _D), lambda i: (i, 0)),
        out_shape=jax.ShapeDtypeStruct((_N, _D), _f32),
    )(s0, s1, h1s, dinvb, b1, W2)


def _k6_body(s0_ref, s1_ref, h2s_ref, dinv_ref, b2_ref, x_ref, out_ref):
    z = (s0_ref[:, :] + s1_ref[:, :] + h2s_ref[:, :]) * dinv_ref[:, :]
    out_ref[:, :] = z + b2_ref[:, :] + x_ref[:, :]


def _k6(s0, s1, h2s, dinvb, b2, x):
    return pl.pallas_call(
        _k6_body,
        grid=(_GRID,),
        in_specs=[
            pl.BlockSpec((_BLK, _D), lambda i: (i, 0)),
            pl.BlockSpec((_BLK, _D), lambda i: (i, 0)),
            pl.BlockSpec((_BLK, _D), lambda i: (i, 0)),
            pl.BlockSpec((_BLK, _D), lambda i: (i, 0)),
            pl.BlockSpec((1, _D), lambda i: (0, 0)),
            pl.BlockSpec((_BLK, _D), lambda i: (i, 0)),
        ],
        out_specs=pl.BlockSpec((_BLK, _D), lambda i: (i, 0)),
        out_shape=jax.ShapeDtypeStruct((_N, _D), _f32),
    )(s0, s1, h2s, dinvb, b2, x)


def kernel(x, edge_index, W1, b1, W2, b2):
    src = edge_index[0]
    dst = edge_index[1]
    pad = _EP - _E
    srcp = jnp.concatenate([src, jnp.zeros((pad,), jnp.int32)])
    dstp = jnp.concatenate([dst, jnp.full((pad,), _N, jnp.int32)])

    degp = _deg_kernel(dstp)
    deg2 = jnp.stack([degp[:_N, 0], degp[_NP:_NP + _N, 0]], axis=1)

    h1s, dinvb = _k2(x, W1, deg2)

    s1 = _agg_kernel(h1s, srcp, dstp)
    h2s = _k4(s1[:_N], s1[_NP:_NP + _N], h1s, dinvb,
              b1.reshape(1, _D), W2)

    s2 = _agg_kernel(h2s, srcp, dstp)
    return _k6(s2[:_N], s2[_NP:_NP + _N], h2s, dinvb,
               b2.reshape(1, _D), x)


# speedup vs baseline: 13.1381x; 1.9889x over previous
import functools

import jax
import jax.numpy as jnp
from jax import lax
from jax.experimental import pallas as pl
from jax.experimental.pallas import tpu as pltpu
from jax.experimental.pallas import tpu_sc as plsc

_N = 10000
_E = 320000
_D = 128

_NC = 2
_NS = 16
_NW = _NC * _NS
_CH = 128
_CPW = 80
_EP = _NW * _CPW * _CH
_RPT = 640
_NP = _NS * _RPT

_f32 = jnp.float32


def _sc_mesh():
    return plsc.VectorSubcoreMesh(core_axis_name="c", subcore_axis_name="s",
                                  num_cores=_NC, num_subcores=_NS)


def _fill_buf(buf, value):
    def _row(i, _):
        r = i // 8
        col = (i % 8) * 16
        buf[r, pl.ds(col, 16)] = jnp.full((16,), value, _f32)
        return _
    lax.fori_loop(0, _CH * 8, _row, None)


def _deg_body(dstp, out, buf, idx_d, acc):
    c = lax.axis_index("c")
    s = lax.axis_index("s")
    w = c * _NS + s

    _fill_buf(buf, 0.0)
    for j in range(_RPT // _CH):
        pltpu.sync_copy(buf, acc.at[pl.ds(s * _RPT + j * _CH, _CH)])
    plsc.subcore_barrier()

    _fill_buf(buf, 1.0)

    base = w * _CPW * _CH

    def _chunk(k, _):
        off = base + k * _CH
        pltpu.sync_copy(dstp.at[pl.ds(off, _CH)], idx_d)
        pltpu.sync_copy(buf, acc.at[idx_d], add=True)
        return _
    lax.fori_loop(0, _CPW, _chunk, None)

    plsc.subcore_barrier()
    pltpu.sync_copy(acc.at[pl.ds(s * _RPT, _RPT)],
                    out.at[pl.ds(c * _NP + s * _RPT, _RPT)])


@functools.partial(
    pl.kernel,
    out_type=jax.ShapeDtypeStruct((_NC * _NP, _D), _f32),
    mesh=_sc_mesh(),
    scratch_types=[
        pltpu.VMEM((_CH, _D), _f32),
        pltpu.VMEM((_CH,), jnp.int32),
        pltpu.VMEM_SHARED((_NP, _D), _f32),
    ],
)
def _deg_kernel(dstp, out, buf, idx_d, acc):
    _deg_body(dstp, out, buf, idx_d, acc)


def _agg_body(tbl, srcp, dstp, out, rows, idx_s, idx_d, acc, sem):
    c = lax.axis_index("c")
    s = lax.axis_index("s")
    w = c * _NS + s

    def _zero_row(i, _):
        r = i // 8
        col = (i % 8) * 16
        rows[r, pl.ds(col, 16)] = jnp.zeros((16,), _f32)
        return _
    lax.fori_loop(0, _CH * 8, _zero_row, None)
    for j in range(_RPT // _CH):
        pltpu.sync_copy(rows, acc.at[pl.ds(s * _RPT + j * _CH, _CH)])
    plsc.subcore_barrier()

    base = w * _CPW * _CH

    def _chunk(k, _):
        off = base + k * _CH
        pltpu.sync_copy(srcp.at[pl.ds(off, _CH)], idx_s)
        pltpu.sync_copy(dstp.at[pl.ds(off, _CH)], idx_d)
        pltpu.async_copy(tbl.at[idx_s], rows, sem).wait()
        pltpu.sync_copy(rows, acc.at[idx_d], add=True)
        return _
    lax.fori_loop(0, _CPW, _chunk, None)

    plsc.subcore_barrier()
    pltpu.sync_copy(acc.at[pl.ds(s * _RPT, _RPT)],
                    out.at[pl.ds(c * _NP + s * _RPT, _RPT)])


@functools.partial(
    pl.kernel,
    out_type=jax.ShapeDtypeStruct((_NC * _NP, _D), _f32),
    mesh=_sc_mesh(),
    scratch_types=[
        pltpu.VMEM((_CH, _D), _f32),
        pltpu.VMEM((_CH,), jnp.int32),
        pltpu.VMEM((_CH,), jnp.int32),
        pltpu.VMEM_SHARED((_NP, _D), _f32),
        pltpu.SemaphoreType.DMA,
    ],
)
def _agg_kernel(tbl, srcp, dstp, out, rows, idx_s, idx_d, acc, sem):
    _agg_body(tbl, srcp, dstp, out, rows, idx_s, idx_d, acc, sem)


_BLK = 1000
_GRID = _N // _BLK


def _k2_body(x_ref, w1_ref, deg_ref, h1s_ref, dinv_ref):
    deg = deg_ref[:, 0:1] + deg_ref[:, 1:2] + 1.0
    dinv = lax.rsqrt(deg)
    dinvb = jnp.broadcast_to(dinv, (_BLK, _D))
    h = jnp.dot(x_ref[:, :], w1_ref[:, :], preferred_element_type=_f32)
    h1s_ref[:, :] = h * dinvb
    dinv_ref[:, :] = dinvb


def _k2(x, W1, deg2):
    return pl.pallas_call(
        _k2_body,
        grid=(_GRID,),
        in_specs=[
            pl.BlockSpec((_BLK, _D), lambda i: (i, 0)),
            pl.BlockSpec((_D, _D), lambda i: (0, 0)),
            pl.BlockSpec((_BLK, 2), lambda i: (i, 0)),
        ],
        out_specs=[
            pl.BlockSpec((_BLK, _D), lambda i: (i, 0)),
            pl.BlockSpec((_BLK, _D), lambda i: (i, 0)),
        ],
        out_shape=[
            jax.ShapeDtypeStruct((_N, _D), _f32),
            jax.ShapeDtypeStruct((_N, _D), _f32),
        ],
    )(x, W1, deg2)


def _k4_body(s0_ref, s1_ref, h1s_ref, dinv_ref, b1_ref, w2_ref, h2s_ref):
    z = (s0_ref[:, :] + s1_ref[:, :] + h1s_ref[:, :]) * dinv_ref[:, :]
    z = z + b1_ref[:, :]
    h2 = jnp.maximum(z, 0.0)
    h2s_ref[:, :] = jnp.dot(h2, w2_ref[:, :],
                            preferred_element_type=_f32) * dinv_ref[:, :]


def _k4(s0, s1, h1s, dinvb, b1, W2):
    return pl.pallas_call(
        _k4_body,
        grid=(_GRID,),
        in_specs=[
            pl.BlockSpec((_BLK, _D), lambda i: (i, 0)),
            pl.BlockSpec((_BLK, _D), lambda i: (i, 0)),
            pl.BlockSpec((_BLK, _D), lambda i: (i, 0)),
            pl.BlockSpec((_BLK, _D), lambda i: (i, 0)),
            pl.BlockSpec((1, _D), lambda i: (0, 0)),
            pl.BlockSpec((_D, _D), lambda i: (0, 0)),
        ],
        out_specs=pl.BlockSpec((_BLK, _D), lambda i: (i, 0)),
        out_shape=jax.ShapeDtypeStruct((_N, _D), _f32),
    )(s0, s1, h1s, dinvb, b1, W2)


def _k6_body(s0_ref, s1_ref, h2s_ref, dinv_ref, b2_ref, x_ref, out_ref):
    z = (s0_ref[:, :] + s1_ref[:, :] + h2s_ref[:, :]) * dinv_ref[:, :]
    out_ref[:, :] = z + b2_ref[:, :] + x_ref[:, :]


def _k6(s0, s1, h2s, dinvb, b2, x):
    return pl.pallas_call(
        _k6_body,
        grid=(_GRID,),
        in_specs=[
            pl.BlockSpec((_BLK, _D), lambda i: (i, 0)),
            pl.BlockSpec((_BLK, _D), lambda i: (i, 0)),
            pl.BlockSpec((_BLK, _D), lambda i: (i, 0)),
            pl.BlockSpec((_BLK, _D), lambda i: (i, 0)),
            pl.BlockSpec((1, _D), lambda i: (0, 0)),
            pl.BlockSpec((_BLK, _D), lambda i: (i, 0)),
        ],
        out_specs=pl.BlockSpec((_BLK, _D), lambda i: (i, 0)),
        out_shape=jax.ShapeDtypeStruct((_N, _D), _f32),
    )(s0, s1, h2s, dinvb, b2, x)


def kernel(x, edge_index, W1, b1, W2, b2):
    src = edge_index[0]
    dst = edge_index[1]
    pad = _EP - _E
    pad_iota = lax.iota(jnp.int32, pad)
    srcp = jnp.concatenate([src, pad_iota % _N])
    dstp = jnp.concatenate([dst, _N + pad_iota % (_NP - _N)])

    degp = _deg_kernel(dstp)
    deg2 = jnp.stack([degp[:_N, 0], degp[_NP:_NP + _N, 0]], axis=1)

    h1s, dinvb = _k2(x, W1, deg2)

    s1 = _agg_kernel(h1s, srcp, dstp)
    h2s = _k4(s1[:_N], s1[_NP:_NP + _N], h1s, dinvb,
              b1.reshape(1, _D), W2)

    s2 = _agg_kernel(h2s, srcp, dstp)
    return _k6(s2[:_N], s2[_NP:_NP + _N], h2s, dinvb,
               b2.reshape(1, _D), x)


# speedup vs baseline: 18.3040x; 1.3932x over previous
import functools

import jax
import jax.numpy as jnp
from jax import lax
from jax.experimental import pallas as pl
from jax.experimental.pallas import tpu as pltpu
from jax.experimental.pallas import tpu_sc as plsc

_N = 10000
_E = 320000
_D = 128

_NC = 2
_NS = 16
_NW = _NC * _NS
_CH = 128
_CPW = 80
_EP = _NW * _CPW * _CH
_RPT = 640
_NP = _NS * _RPT

_f32 = jnp.float32


def _sc_mesh():
    return plsc.VectorSubcoreMesh(core_axis_name="c", subcore_axis_name="s",
                                  num_cores=_NC, num_subcores=_NS)


def _fill_buf(buf, value):
    def _row(i, _):
        r = i // 8
        col = (i % 8) * 16
        buf[r, pl.ds(col, 16)] = jnp.full((16,), value, _f32)
        return _
    lax.fori_loop(0, _CH * 8, _row, None)


def _deg_body(dstp, out, buf, idx_d, acc):
    c = lax.axis_index("c")
    s = lax.axis_index("s")
    w = c * _NS + s

    _fill_buf(buf, 0.0)
    for j in range(_RPT // _CH):
        pltpu.sync_copy(buf, acc.at[pl.ds(s * _RPT + j * _CH, _CH)])
    plsc.subcore_barrier()

    _fill_buf(buf, 1.0)

    base = w * _CPW * _CH

    def _chunk(k, _):
        off = base + k * _CH
        pltpu.sync_copy(dstp.at[pl.ds(off, _CH)], idx_d)
        pltpu.sync_copy(buf, acc.at[idx_d], add=True)
        return _
    lax.fori_loop(0, _CPW, _chunk, None)

    plsc.subcore_barrier()
    pltpu.sync_copy(acc.at[pl.ds(s * _RPT, _RPT)],
                    out.at[pl.ds(c * _NP + s * _RPT, _RPT)])


@functools.partial(
    pl.kernel,
    out_type=jax.ShapeDtypeStruct((_NC * _NP, _D), _f32),
    mesh=_sc_mesh(),
    scratch_types=[
        pltpu.VMEM((_CH, _D), _f32),
        pltpu.VMEM((_CH,), jnp.int32),
        pltpu.VMEM_SHARED((_NP, _D), _f32),
    ],
)
def _deg_kernel(dstp, out, buf, idx_d, acc):
    _deg_body(dstp, out, buf, idx_d, acc)


def _agg_body(tbl, srcp, dstp, out, rows0, rows1, idx_s0, idx_d0,
              idx_s1, idx_d1, acc, sem0, sem1):
    c = lax.axis_index("c")
    s = lax.axis_index("s")
    w = c * _NS + s

    def _zero_row(i, _):
        r = i // 8
        col = (i % 8) * 16
        rows0[r, pl.ds(col, 16)] = jnp.zeros((16,), _f32)
        return _
    lax.fori_loop(0, _CH * 8, _zero_row, None)
    for j in range(_RPT // _CH):
        pltpu.sync_copy(rows0, acc.at[pl.ds(s * _RPT + j * _CH, _CH)])
    plsc.subcore_barrier()

    base = w * _CPW * _CH

    def _load_idx(k, idx_s, idx_d):
        off = base + k * _CH
        pltpu.sync_copy(srcp.at[pl.ds(off, _CH)], idx_s)
        pltpu.sync_copy(dstp.at[pl.ds(off, _CH)], idx_d)

    _load_idx(0, idx_s0, idx_d0)
    cp0 = pltpu.async_copy(tbl.at[idx_s0], rows0, sem0)
    _load_idx(1, idx_s1, idx_d1)
    cp1 = pltpu.async_copy(tbl.at[idx_s1], rows1, sem1)

    def _pair(i, _):
        k = 2 * i
        cp0.wait()
        pltpu.sync_copy(rows0, acc.at[idx_d0], add=True)
        _load_idx(k + 2, idx_s0, idx_d0)
        pltpu.async_copy(tbl.at[idx_s0], rows0, sem0)
        cp1.wait()
        pltpu.sync_copy(rows1, acc.at[idx_d1], add=True)
        _load_idx(k + 3, idx_s1, idx_d1)
        pltpu.async_copy(tbl.at[idx_s1], rows1, sem1)
        return _
    lax.fori_loop(0, _CPW // 2 - 1, _pair, None)

    cp0.wait()
    pltpu.sync_copy(rows0, acc.at[idx_d0], add=True)
    cp1.wait()
    pltpu.sync_copy(rows1, acc.at[idx_d1], add=True)

    plsc.subcore_barrier()
    pltpu.sync_copy(acc.at[pl.ds(s * _RPT, _RPT)],
                    out.at[pl.ds(c * _NP + s * _RPT, _RPT)])


@functools.partial(
    pl.kernel,
    out_type=jax.ShapeDtypeStruct((_NC * _NP, _D), _f32),
    mesh=_sc_mesh(),
    scratch_types=[
        pltpu.VMEM((_CH, _D), _f32),
        pltpu.VMEM((_CH, _D), _f32),
        pltpu.VMEM((_CH,), jnp.int32),
        pltpu.VMEM((_CH,), jnp.int32),
        pltpu.VMEM((_CH,), jnp.int32),
        pltpu.VMEM((_CH,), jnp.int32),
        pltpu.VMEM_SHARED((_NP, _D), _f32),
        pltpu.SemaphoreType.DMA,
        pltpu.SemaphoreType.DMA,
    ],
)
def _agg_kernel(tbl, srcp, dstp, out, rows0, rows1, idx_s0, idx_d0,
                idx_s1, idx_d1, acc, sem0, sem1):
    _agg_body(tbl, srcp, dstp, out, rows0, rows1, idx_s0, idx_d0,
              idx_s1, idx_d1, acc, sem0, sem1)


_BLK = 1000
_GRID = _N // _BLK


def _k2_body(x_ref, w1_ref, deg_ref, h1s_ref, dinv_ref):
    deg = deg_ref[:, 0:1] + deg_ref[:, 1:2] + 1.0
    dinv = lax.rsqrt(deg)
    dinvb = jnp.broadcast_to(dinv, (_BLK, _D))
    h = jnp.dot(x_ref[:, :], w1_ref[:, :], preferred_element_type=_f32)
    h1s_ref[:, :] = h * dinvb
    dinv_ref[:, :] = dinvb


def _k2(x, W1, deg2):
    return pl.pallas_call(
        _k2_body,
        grid=(_GRID,),
        in_specs=[
            pl.BlockSpec((_BLK, _D), lambda i: (i, 0)),
            pl.BlockSpec((_D, _D), lambda i: (0, 0)),
            pl.BlockSpec((_BLK, 2), lambda i: (i, 0)),
        ],
        out_specs=[
            pl.BlockSpec((_BLK, _D), lambda i: (i, 0)),
            pl.BlockSpec((_BLK, _D), lambda i: (i, 0)),
        ],
        out_shape=[
            jax.ShapeDtypeStruct((_N, _D), _f32),
            jax.ShapeDtypeStruct((_N, _D), _f32),
        ],
    )(x, W1, deg2)


def _k4_body(s0_ref, s1_ref, h1s_ref, dinv_ref, b1_ref, w2_ref, h2s_ref):
    z = (s0_ref[:, :] + s1_ref[:, :] + h1s_ref[:, :]) * dinv_ref[:, :]
    z = z + b1_ref[:, :]
    h2 = jnp.maximum(z, 0.0)
    h2s_ref[:, :] = jnp.dot(h2, w2_ref[:, :],
                            preferred_element_type=_f32) * dinv_ref[:, :]


def _k4(s0, s1, h1s, dinvb, b1, W2):
    return pl.pallas_call(
        _k4_body,
        grid=(_GRID,),
        in_specs=[
            pl.BlockSpec((_BLK, _D), lambda i: (i, 0)),
            pl.BlockSpec((_BLK, _D), lambda i: (i, 0)),
            pl.BlockSpec((_BLK, _D), lambda i: (i, 0)),
            pl.BlockSpec((_BLK, _D), lambda i: (i, 0)),
            pl.BlockSpec((1, _D), lambda i: (0, 0)),
            pl.BlockSpec((_D, _D), lambda i: (0, 0)),
        ],
        out_specs=pl.BlockSpec((_BLK, _D), lambda i: (i, 0)),
        out_shape=jax.ShapeDtypeStruct((_N, _D), _f32),
    )(s0, s1, h1s, dinvb, b1, W2)


def _k6_body(s0_ref, s1_ref, h2s_ref, dinv_ref, b2_ref, x_ref, out_ref):
    z = (s0_ref[:, :] + s1_ref[:, :] + h2s_ref[:, :]) * dinv_ref[:, :]
    out_ref[:, :] = z + b2_ref[:, :] + x_ref[:, :]


def _k6(s0, s1, h2s, dinvb, b2, x):
    return pl.pallas_call(
        _k6_body,
        grid=(_GRID,),
        in_specs=[
            pl.BlockSpec((_BLK, _D), lambda i: (i, 0)),
            pl.BlockSpec((_BLK, _D), lambda i: (i, 0)),
            pl.BlockSpec((_BLK, _D), lambda i: (i, 0)),
            pl.BlockSpec((_BLK, _D), lambda i: (i, 0)),
            pl.BlockSpec((1, _D), lambda i: (0, 0)),
            pl.BlockSpec((_BLK, _D), lambda i: (i, 0)),
        ],
        out_specs=pl.BlockSpec((_BLK, _D), lambda i: (i, 0)),
        out_shape=jax.ShapeDtypeStruct((_N, _D), _f32),
    )(s0, s1, h2s, dinvb, b2, x)


def kernel(x, edge_index, W1, b1, W2, b2):
    src = edge_index[0]
    dst = edge_index[1]
    pad = _EP - _E
    pad_iota = lax.iota(jnp.int32, pad)
    srcp = jnp.concatenate([src, pad_iota % _N])
    dstp = jnp.concatenate([dst, _N + pad_iota % (_NP - _N)])

    degp = _deg_kernel(dstp)
    deg2 = jnp.stack([degp[:_N, 0], degp[_NP:_NP + _N, 0]], axis=1)

    h1s, dinvb = _k2(x, W1, deg2)

    s1 = _agg_kernel(h1s, srcp, dstp)
    h2s = _k4(s1[:_N], s1[_NP:_NP + _N], h1s, dinvb,
              b1.reshape(1, _D), W2)

    s2 = _agg_kernel(h2s, srcp, dstp)
    return _k6(s2[:_N], s2[_NP:_NP + _N], h2s, dinvb,
               b2.reshape(1, _D), x)


# speedup vs baseline: 18.7632x; 1.0251x over previous
import functools

import jax
import jax.numpy as jnp
from jax import lax
from jax.experimental import pallas as pl
from jax.experimental.pallas import tpu as pltpu
from jax.experimental.pallas import tpu_sc as plsc

_N = 10000
_E = 320000
_D = 128

_NC = 2
_NS = 16
_NW = _NC * _NS
_CH = 112
_CPW = 90
_EP = _NW * _CPW * _CH
_RPT = 632
_NP = _NS * _RPT

_f32 = jnp.float32


def _sc_mesh():
    return plsc.VectorSubcoreMesh(core_axis_name="c", subcore_axis_name="s",
                                  num_cores=_NC, num_subcores=_NS)


def _zero_acc_slice(buf, acc, s):
    for j in range(_RPT // _CH):
        pltpu.sync_copy(buf, acc.at[pl.ds(s * _RPT + j * _CH, _CH)])
    rem = _RPT % _CH
    if rem:
        pltpu.sync_copy(buf.at[pl.ds(0, rem)],
                        acc.at[pl.ds(s * _RPT + (_RPT // _CH) * _CH, rem)])


def _fill_buf(buf, value):
    def _row(i, _):
        r = i // 8
        col = (i % 8) * 16
        buf[r, pl.ds(col, 16)] = jnp.full((16,), value, _f32)
        return _
    lax.fori_loop(0, _CH * 8, _row, None)


def _deg_body(dstp, out, buf, idx_d, acc):
    c = lax.axis_index("c")
    s = lax.axis_index("s")
    w = c * _NS + s

    _fill_buf(buf, 0.0)
    _zero_acc_slice(buf, acc, s)
    plsc.subcore_barrier()

    _fill_buf(buf, 1.0)

    base = w * _CPW * _CH

    def _chunk(k, _):
        off = base + k * _CH
        pltpu.sync_copy(dstp.at[pl.ds(off, _CH)], idx_d)
        pltpu.sync_copy(buf, acc.at[idx_d], add=True)
        return _
    lax.fori_loop(0, _CPW, _chunk, None)

    plsc.subcore_barrier()
    pltpu.sync_copy(acc.at[pl.ds(s * _RPT, _RPT)],
                    out.at[pl.ds(c * _NP + s * _RPT, _RPT)])


@functools.partial(
    pl.kernel,
    out_type=jax.ShapeDtypeStruct((_NC * _NP, _D), _f32),
    mesh=_sc_mesh(),
    scratch_types=[
        pltpu.VMEM((_CH, _D), _f32),
        pltpu.VMEM((_CH,), jnp.int32),
        pltpu.VMEM_SHARED((_NP, _D), _f32),
    ],
)
def _deg_kernel(dstp, out, buf, idx_d, acc):
    _deg_body(dstp, out, buf, idx_d, acc)


_NBUF = 3
_NGRP = _CPW // _NBUF


def _agg_body(tbl, srcp, dstp, out, rows, idx_s, idx_d, acc, gsem, ssem):
    c = lax.axis_index("c")
    s = lax.axis_index("s")
    w = c * _NS + s

    _fill_buf(rows[0], 0.0)
    _zero_acc_slice(rows[0], acc, s)
    plsc.subcore_barrier()

    base = w * _CPW * _CH

    def _load_idx(k, b):
        off = base + k * _CH
        pltpu.sync_copy(srcp.at[pl.ds(off, _CH)], idx_s[b])
        pltpu.sync_copy(dstp.at[pl.ds(off, _CH)], idx_d[b])

    gd = [pltpu.make_async_copy(tbl.at[idx_s[b]], rows[b], gsem[b])
          for b in range(_NBUF)]
    sd = [pltpu.make_async_copy(rows[b], acc.at[idx_d[b]], ssem[b])
          for b in range(_NBUF)]

    for b in range(_NBUF):
        _load_idx(b, b)
        gd[b].start()

    def _group(i, _):
        for b in range(_NBUF):
            gd[b].wait()
            sd[b].start(add=True)
        for b in range(_NBUF):
            sd[b].wait()
            _load_idx((i + 1) * _NBUF + b, b)
            gd[b].start()
        return _
    lax.fori_loop(0, _NGRP - 1, _group, None)

    for b in range(_NBUF):
        gd[b].wait()
        sd[b].start(add=True)
    for b in range(_NBUF):
        sd[b].wait()

    plsc.subcore_barrier()
    pltpu.sync_copy(acc.at[pl.ds(s * _RPT, _RPT)],
                    out.at[pl.ds(c * _NP + s * _RPT, _RPT)])


@functools.partial(
    pl.kernel,
    out_type=jax.ShapeDtypeStruct((_NC * _NP, _D), _f32),
    mesh=_sc_mesh(),
    scratch_types=[
        [pltpu.VMEM((_CH, _D), _f32)] * _NBUF,
        [pltpu.VMEM((_CH,), jnp.int32)] * _NBUF,
        [pltpu.VMEM((_CH,), jnp.int32)] * _NBUF,
        pltpu.VMEM_SHARED((_NP, _D), _f32),
        [pltpu.SemaphoreType.DMA] * _NBUF,
        [pltpu.SemaphoreType.DMA] * _NBUF,
    ],
)
def _agg_kernel(tbl, srcp, dstp, out, rows, idx_s, idx_d, acc, gsem, ssem):
    _agg_body(tbl, srcp, dstp, out, rows, idx_s, idx_d, acc, gsem, ssem)


_BLK = 1000
_GRID = _N // _BLK


def _k2_body(x_ref, w1_ref, deg_ref, h1s_ref, dinv_ref):
    deg = deg_ref[:, 0:1] + deg_ref[:, 1:2] + 1.0
    dinv = lax.rsqrt(deg)
    dinvb = jnp.broadcast_to(dinv, (_BLK, _D))
    h = jnp.dot(x_ref[:, :], w1_ref[:, :], preferred_element_type=_f32)
    h1s_ref[:, :] = h * dinvb
    dinv_ref[:, :] = dinvb


def _k2(x, W1, deg2):
    return pl.pallas_call(
        _k2_body,
        grid=(_GRID,),
        in_specs=[
            pl.BlockSpec((_BLK, _D), lambda i: (i, 0)),
            pl.BlockSpec((_D, _D), lambda i: (0, 0)),
            pl.BlockSpec((_BLK, 2), lambda i: (i, 0)),
        ],
        out_specs=[
            pl.BlockSpec((_BLK, _D), lambda i: (i, 0)),
            pl.BlockSpec((_BLK, _D), lambda i: (i, 0)),
        ],
        out_shape=[
            jax.ShapeDtypeStruct((_N, _D), _f32),
            jax.ShapeDtypeStruct((_N, _D), _f32),
        ],
    )(x, W1, deg2)


def _k4_body(s0_ref, s1_ref, h1s_ref, dinv_ref, b1_ref, w2_ref, h2s_ref):
    z = (s0_ref[:, :] + s1_ref[:, :] + h1s_ref[:, :]) * dinv_ref[:, :]
    z = z + b1_ref[:, :]
    h2 = jnp.maximum(z, 0.0)
    h2s_ref[:, :] = jnp.dot(h2, w2_ref[:, :],
                            preferred_element_type=_f32) * dinv_ref[:, :]


def _k4(s0, s1, h1s, dinvb, b1, W2):
    return pl.pallas_call(
        _k4_body,
        grid=(_GRID,),
        in_specs=[
            pl.BlockSpec((_BLK, _D), lambda i: (i, 0)),
            pl.BlockSpec((_BLK, _D), lambda i: (i, 0)),
            pl.BlockSpec((_BLK, _D), lambda i: (i, 0)),
            pl.BlockSpec((_BLK, _D), lambda i: (i, 0)),
            pl.BlockSpec((1, _D), lambda i: (0, 0)),
            pl.BlockSpec((_D, _D), lambda i: (0, 0)),
        ],
        out_specs=pl.BlockSpec((_BLK, _D), lambda i: (i, 0)),
        out_shape=jax.ShapeDtypeStruct((_N, _D), _f32),
    )(s0, s1, h1s, dinvb, b1, W2)


def _k6_body(s0_ref, s1_ref, h2s_ref, dinv_ref, b2_ref, x_ref, out_ref):
    z = (s0_ref[:, :] + s1_ref[:, :] + h2s_ref[:, :]) * dinv_ref[:, :]
    out_ref[:, :] = z + b2_ref[:, :] + x_ref[:, :]


def _k6(s0, s1, h2s, dinvb, b2, x):
    return pl.pallas_call(
        _k6_body,
        grid=(_GRID,),
        in_specs=[
            pl.BlockSpec((_BLK, _D), lambda i: (i, 0)),
            pl.BlockSpec((_BLK, _D), lambda i: (i, 0)),
            pl.BlockSpec((_BLK, _D), lambda i: (i, 0)),
            pl.BlockSpec((_BLK, _D), lambda i: (i, 0)),
            pl.BlockSpec((1, _D), lambda i: (0, 0)),
            pl.BlockSpec((_BLK, _D), lambda i: (i, 0)),
        ],
        out_specs=pl.BlockSpec((_BLK, _D), lambda i: (i, 0)),
        out_shape=jax.ShapeDtypeStruct((_N, _D), _f32),
    )(s0, s1, h2s, dinvb, b2, x)


def kernel(x, edge_index, W1, b1, W2, b2):
    src = edge_index[0]
    dst = edge_index[1]
    pad = _EP - _E
    pad_iota = lax.iota(jnp.int32, pad)
    srcp = jnp.concatenate([src, pad_iota % _N])
    dstp = jnp.concatenate([dst, _N + pad_iota % (_NP - _N)])

    degp = _deg_kernel(dstp)
    deg2 = jnp.stack([degp[:_N, 0], degp[_NP:_NP + _N, 0]], axis=1)

    h1s, dinvb = _k2(x, W1, deg2)

    s1 = _agg_kernel(h1s, srcp, dstp)
    h2s = _k4(s1[:_N], s1[_NP:_NP + _N], h1s, dinvb,
              b1.reshape(1, _D), W2)

    s2 = _agg_kernel(h2s, srcp, dstp)
    return _k6(s2[:_N], s2[_NP:_NP + _N], h2s, dinvb,
               b2.reshape(1, _D), x)
